# baseline (device time: 11004 ns/iter reference)
import jax
import jax.numpy as jnp
from jax import lax
from jax.experimental import pallas as pl
from jax.experimental.pallas import tpu as pltpu

N_DEV = 4
K = 8


def _peer_rdmas(my, src_ref, comm_ref, send_sems, recv_sems, stage):
    rdmas = []
    for off in range(1, N_DEV):
        slot = N_DEV - 1 - off
        rdmas.append(
            pltpu.make_async_remote_copy(
                src_ref=src_ref,
                dst_ref=comm_ref.at[stage, slot],
                send_sem=send_sems.at[stage, off - 1],
                recv_sem=recv_sems.at[stage, slot],
                device_id=((my + off) % N_DEV,),
                device_id_type=pl.DeviceIdType.MESH,
            )
        )
    return rdmas


def kernel(x):
    m_per, n = x.shape
    total_rows = N_DEV * m_per
    rows_blk = m_per // K

    def body(x_hbm, out_ref, buf, acc_a, acc_b, comm_ref, copy_sems,
             send_sems, recv_sems):
        my = lax.axis_index("i")
        barrier_sem = pltpu.get_barrier_semaphore()
        for off in range(1, N_DEV):
            pl.semaphore_signal(
                barrier_sem, inc=1,
                device_id=((my + off) % N_DEV,),
                device_id_type=pl.DeviceIdType.MESH,
            )

        def block_copy(k, slot):
            return pltpu.make_async_copy(
                x_hbm.at[pl.ds(k * rows_blk, rows_blk), :],
                buf.at[slot],
                copy_sems.at[slot],
            )

        block_copy(0, 0).start()
        for k in range(K):
            slot = k % 2
            if k + 1 < K:
                block_copy(k + 1, (k + 1) % 2).start()
            block_copy(k, slot).wait()
            blk = jnp.sum(buf[slot], axis=0, keepdims=True)
            if k == 0:
                acc_a[:, :] = blk
            elif k < K - 1:
                acc_a[:, :] = acc_a[:, :] + blk
            else:
                acc_b[:, :] = blk

            if k == K - 2:
                pl.semaphore_wait(barrier_sem, N_DEV - 1)
                for rdma in _peer_rdmas(my, acc_a, comm_ref, send_sems,
                                        recv_sems, 0):
                    rdma.start()

        rdmas_b = _peer_rdmas(my, acc_b, comm_ref, send_sems, recv_sems, 1)
        for rdma in rdmas_b:
            rdma.start()
        rdmas_a = _peer_rdmas(my, acc_a, comm_ref, send_sems, recv_sems, 0)
        for rdma in rdmas_a + rdmas_b:
            rdma.wait()

        acc = acc_a[:, :] + acc_b[:, :]
        for stage in range(2):
            for slot in range(N_DEV - 1):
                acc = acc + comm_ref[stage, slot, :, :]
        out_ref[:, :] = acc * (1.0 / total_rows)

    x = pltpu.with_memory_space_constraint(x, pltpu.MemorySpace.HBM)
    return pl.pallas_call(
        body,
        out_shape=jax.ShapeDtypeStruct((1, n), x.dtype),
        in_specs=[pl.BlockSpec(memory_space=pltpu.MemorySpace.HBM)],
        out_specs=pl.BlockSpec(memory_space=pltpu.VMEM),
        scratch_shapes=[
            pltpu.VMEM((2, rows_blk, n), x.dtype),
            pltpu.VMEM((1, n), x.dtype),
            pltpu.VMEM((1, n), x.dtype),
            pltpu.VMEM((2, N_DEV - 1, 1, n), x.dtype),
            pltpu.SemaphoreType.DMA((2,)),
            pltpu.SemaphoreType.DMA((2, N_DEV - 1)),
            pltpu.SemaphoreType.DMA((2, N_DEV - 1)),
        ],
        compiler_params=pltpu.CompilerParams(collective_id=0),
    )(x)


# device time: 8841 ns/iter; 1.2447x vs baseline; 1.2447x over previous
import jax
import jax.numpy as jnp
from jax import lax
from jax.experimental import pallas as pl
from jax.experimental.pallas import tpu as pltpu

N_DEV = 4
K = 8


def _peer_rdmas(my, src_ref, comm_ref, send_sems, recv_sems, stage):
    rdmas = []
    for off in range(1, N_DEV):
        slot = N_DEV - 1 - off
        rdmas.append(
            pltpu.make_async_remote_copy(
                src_ref=src_ref,
                dst_ref=comm_ref.at[stage, slot],
                send_sem=send_sems.at[stage, off - 1],
                recv_sem=recv_sems.at[stage, slot],
                device_id=((my + off) % N_DEV,),
                device_id_type=pl.DeviceIdType.MESH,
            )
        )
    return rdmas


def kernel(x):
    m_per, n = x.shape
    total_rows = N_DEV * m_per
    rows_blk = m_per // K

    def body(x_hbm, out_ref, buf, acc_a, acc_b, comm_ref, copy_sems,
             send_sems, recv_sems):
        my = lax.axis_index("i")
        barrier_sem = pltpu.get_barrier_semaphore()
        for off in range(1, N_DEV):
            pl.semaphore_signal(
                barrier_sem, inc=1,
                device_id=((my + off) % N_DEV,),
                device_id_type=pl.DeviceIdType.MESH,
            )

        def block_copy(k):
            return pltpu.make_async_copy(
                x_hbm.at[pl.ds(k * rows_blk, rows_blk), :],
                buf.at[k],
                copy_sems.at[k],
            )

        for k in range(K):
            block_copy(k).start()
        for k in range(K):
            block_copy(k).wait()
            blk = jnp.sum(buf[k], axis=0, keepdims=True)
            if k == 0:
                acc_a[:, :] = blk
            elif k < K - 1:
                acc_a[:, :] = acc_a[:, :] + blk
            else:
                acc_b[:, :] = blk

            if k == K - 2:
                pl.semaphore_wait(barrier_sem, N_DEV - 1)
                for rdma in _peer_rdmas(my, acc_a, comm_ref, send_sems,
                                        recv_sems, 0):
                    rdma.start()

        rdmas_b = _peer_rdmas(my, acc_b, comm_ref, send_sems, recv_sems, 1)
        for rdma in rdmas_b:
            rdma.start()
        rdmas_a = _peer_rdmas(my, acc_a, comm_ref, send_sems, recv_sems, 0)
        for rdma in rdmas_a + rdmas_b:
            rdma.wait()

        acc = acc_a[:, :] + acc_b[:, :]
        for stage in range(2):
            for slot in range(N_DEV - 1):
                acc = acc + comm_ref[stage, slot, :, :]
        out_ref[:, :] = acc * (1.0 / total_rows)

    x = pltpu.with_memory_space_constraint(x, pltpu.MemorySpace.HBM)
    return pl.pallas_call(
        body,
        out_shape=jax.ShapeDtypeStruct((1, n), x.dtype),
        in_specs=[pl.BlockSpec(memory_space=pltpu.MemorySpace.HBM)],
        out_specs=pl.BlockSpec(memory_space=pltpu.VMEM),
        scratch_shapes=[
            pltpu.VMEM((K, rows_blk, n), x.dtype),
            pltpu.VMEM((1, n), x.dtype),
            pltpu.VMEM((1, n), x.dtype),
            pltpu.VMEM((2, N_DEV - 1, 1, n), x.dtype),
            pltpu.SemaphoreType.DMA((K,)),
            pltpu.SemaphoreType.DMA((2, N_DEV - 1)),
            pltpu.SemaphoreType.DMA((2, N_DEV - 1)),
        ],
        compiler_params=pltpu.CompilerParams(collective_id=0),
    )(x)


# device time: 8784 ns/iter; 1.2527x vs baseline; 1.0065x over previous
import jax
import jax.numpy as jnp
from jax import lax
from jax.experimental import pallas as pl
from jax.experimental.pallas import tpu as pltpu

N_DEV = 4
K = 8


def kernel(x):
    m_per, n = x.shape
    total_rows = N_DEV * m_per
    rows_blk = m_per // K

    def body(x_hbm, out_ref, buf, acc, comm_ref, copy_sems, send_sems,
             recv_sems):
        my = lax.axis_index("i")
        barrier_sem = pltpu.get_barrier_semaphore()
        for off in range(1, N_DEV):
            pl.semaphore_signal(
                barrier_sem, inc=1,
                device_id=((my + off) % N_DEV,),
                device_id_type=pl.DeviceIdType.MESH,
            )

        def block_copy(k):
            return pltpu.make_async_copy(
                x_hbm.at[pl.ds(k * rows_blk, rows_blk), :],
                buf.at[k],
                copy_sems.at[k],
            )

        for k in range(K):
            block_copy(k).start()
        for k in range(K):
            block_copy(k).wait()
            blk = jnp.sum(buf[k], axis=0, keepdims=True)
            if k == 0:
                acc[:, :] = blk
            else:
                acc[:, :] = acc[:, :] + blk

        pl.semaphore_wait(barrier_sem, N_DEV - 1)

        rdmas = []
        for off in range(1, N_DEV):
            slot = N_DEV - 1 - off
            rdma = pltpu.make_async_remote_copy(
                src_ref=acc,
                dst_ref=comm_ref.at[slot],
                send_sem=send_sems.at[off - 1],
                recv_sem=recv_sems.at[slot],
                device_id=((my + off) % N_DEV,),
                device_id_type=pl.DeviceIdType.MESH,
            )
            rdma.start()
            rdmas.append(rdma)
        for rdma in rdmas:
            rdma.wait()

        total = acc[:, :]
        for slot in range(N_DEV - 1):
            total = total + comm_ref[slot, :, :]
        out_ref[:, :] = total * (1.0 / total_rows)

    x = pltpu.with_memory_space_constraint(x, pltpu.MemorySpace.HBM)
    return pl.pallas_call(
        body,
        out_shape=jax.ShapeDtypeStruct((1, n), x.dtype),
        in_specs=[pl.BlockSpec(memory_space=pltpu.MemorySpace.HBM)],
        out_specs=pl.BlockSpec(memory_space=pltpu.VMEM),
        scratch_shapes=[
            pltpu.VMEM((K, rows_blk, n), x.dtype),
            pltpu.VMEM((1, n), x.dtype),
            pltpu.VMEM((N_DEV - 1, 1, n), x.dtype),
            pltpu.SemaphoreType.DMA((K,)),
            pltpu.SemaphoreType.DMA((N_DEV - 1,)),
            pltpu.SemaphoreType.DMA((N_DEV - 1,)),
        ],
        compiler_params=pltpu.CompilerParams(collective_id=0),
    )(x)
